# baseline (device time: 19043 ns/iter reference)
import os

import jax
import jax.numpy as jnp
from jax import lax
from jax.experimental import pallas as pl
from jax.experimental.pallas import tpu as pltpu

_NO_COMM = os.environ.get("VEMBED_NO_COMM", "0") == "1"

N_DEV = 4
V_PER = 4096
T = 512
TB = T // N_DEV
D = 512
NH = 4
DH = D // NH


def kernel(ids, E):
    ids2 = jnp.reshape(ids, (T, 1))

    def body(ids_ref, e_ref, out_ref, ids_ext,
             rs_send_buf, rs_buf, ag_send_buf, ag_buf,
             rs_send_sems, rs_recv_sems, ag_send_sems, ag_recv_sems):
        my_pos = lax.axis_index("i")

        ids_ext[0:T, :] = ids_ref[:, :]
        ids_ext[T:2 * T, :] = ids_ref[:, :]

        if not _NO_COMM:
            barrier_sem = pltpu.get_barrier_semaphore()
            for d in range(1, N_DEV):
                pl.semaphore_signal(
                    barrier_sem, inc=1,
                    device_id=((my_pos + d) % N_DEV,),
                    device_id_type=pl.DeviceIdType.MESH,
                )
            pl.semaphore_wait(barrier_sem, N_DEV - 1)

        base = my_pos * V_PER
        iota = lax.broadcasted_iota(jnp.int32, (2 * TB, V_PER), 1)
        e_bf = e_ref[:, :].astype(jnp.bfloat16)

        def partial_pair(start):
            loc = ids_ext[pl.ds(start, 2 * TB), :] - base
            onehot = (iota == loc).astype(jnp.bfloat16)
            return jnp.dot(onehot, e_bf,
                           preferred_element_type=jnp.float32
                           ).astype(jnp.bfloat16)

        def start_rs(d, rows):
            peer = (my_pos + d) % N_DEV
            for h in range(NH):
                rs_send_buf[h, d - 1, :, :] = rows[:, h * DH:(h + 1) * DH]
                rdma = pltpu.make_async_remote_copy(
                    src_ref=rs_send_buf.at[h, d - 1],
                    dst_ref=rs_buf.at[h, d - 1],
                    send_sem=rs_send_sems.at[h, d - 1],
                    recv_sem=rs_recv_sems.at[h, d - 1],
                    device_id=(peer,),
                    device_id_type=pl.DeviceIdType.MESH,
                )
                rdma.start()
                rs_rdmas[h, d] = rdma

        rs_rdmas = {}
        pa = partial_pair((my_pos + 1) * TB)
        if not _NO_COMM:
            start_rs(2, pa[TB:, :])
            start_rs(1, pa[:TB, :])
        pb = partial_pair((my_pos + 3) * TB)

        if _NO_COMM:
            out_ref[0:2 * TB, :] = pa
            out_ref[2 * TB:, :] = pb
            return

        start_rs(3, pb[:TB, :])

        own = pb[TB:, :]
        ag_rdmas = {}
        for h in range(NH):
            cols = pl.ds(h * DH, DH)
            acc_h = own[:, h * DH:(h + 1) * DH]
            for d in (1, 3, 2):
                rs_rdmas[h, d].wait_recv()
                acc_h = acc_h + rs_buf[h, d - 1, :, :]
            ag_send_buf[h, :, :] = acc_h
            for d in (2, 1, 3):
                peer = (my_pos + d) % N_DEV
                rdma = pltpu.make_async_remote_copy(
                    src_ref=ag_send_buf.at[h],
                    dst_ref=ag_buf.at[h, d - 1],
                    send_sem=ag_send_sems.at[h, d - 1],
                    recv_sem=ag_recv_sems.at[h, d - 1],
                    device_id=(peer,),
                    device_id_type=pl.DeviceIdType.MESH,
                )
                rdma.start()
                ag_rdmas[h, d] = rdma
            out_ref[pl.ds(my_pos * TB, TB), cols] = acc_h

        for h in range(NH):
            cols = pl.ds(h * DH, DH)
            for d in (1, 3, 2):
                src = (my_pos - d) % N_DEV
                ag_rdmas[h, d].wait_recv()
                out_ref[pl.ds(src * TB, TB), cols] = ag_buf[h, d - 1, :, :]

        for h in range(NH):
            for d in range(1, N_DEV):
                rs_rdmas[h, d].wait_send()
                ag_rdmas[h, d].wait_send()

    return pl.pallas_call(
        body,
        out_shape=jax.ShapeDtypeStruct((T, D), jnp.bfloat16),
        in_specs=[
            pl.BlockSpec(memory_space=pltpu.VMEM),
            pl.BlockSpec(memory_space=pltpu.VMEM),
        ],
        out_specs=pl.BlockSpec(memory_space=pltpu.VMEM),
        scratch_shapes=[
            pltpu.VMEM((2 * T, 1), jnp.int32),
            pltpu.VMEM((NH, N_DEV - 1, TB, DH), jnp.bfloat16),
            pltpu.VMEM((NH, N_DEV - 1, TB, DH), jnp.bfloat16),
            pltpu.VMEM((NH, TB, DH), jnp.bfloat16),
            pltpu.VMEM((NH, N_DEV - 1, TB, DH), jnp.bfloat16),
            pltpu.SemaphoreType.DMA((NH, N_DEV - 1)),
            pltpu.SemaphoreType.DMA((NH, N_DEV - 1)),
            pltpu.SemaphoreType.DMA((NH, N_DEV - 1)),
            pltpu.SemaphoreType.DMA((NH, N_DEV - 1)),
        ],
        compiler_params=(None if _NO_COMM
                         else pltpu.CompilerParams(collective_id=0)),
    )(ids2, E)
